# Initial kernel scaffold; baseline (speedup 1.0000x reference)
import functools

import jax
import jax.numpy as jnp
from jax import lax
from jax.experimental import pallas as pl
from jax.experimental.pallas import tpu as pltpu
from jax.experimental.pallas import tpu_sc as plsc

N_NODES = 10000
N_EDGES = 320000
NC, NS = 2, 16
NW = NC * NS
EPW = N_EDGES // NW
K = 80
NCHUNK = EPW // K
RPT = N_NODES // NS

_MESH = plsc.VectorSubcoreMesh(core_axis_name="c", subcore_axis_name="s")


def _make_deg_kernel():
  D = 16

  @functools.partial(
      pl.kernel,
      out_type=jax.ShapeDtypeStruct((NC, N_NODES, D), jnp.float32),
      mesh=_MESH,
      scratch_types=[
          pltpu.VMEM((K,), jnp.int32),
          pltpu.VMEM((K, D), jnp.float32),
          pltpu.VMEM_SHARED((N_NODES, D), jnp.float32),
      ],
  )
  def deg_kernel(dst_hbm, zeros_hbm, out_hbm, dst_v, rows_v, acc_sh):
    c = lax.axis_index("c")
    s = lax.axis_index("s")
    wid = c * NS + s
    r0 = s * RPT
    pltpu.sync_copy(zeros_hbm.at[pl.ds(r0, RPT)], acc_sh.at[pl.ds(r0, RPT)])
    ones = jnp.ones((16,), jnp.float32)
    for r in range(K):
      rows_v[r, :] = ones
    plsc.subcore_barrier()
    base = wid * EPW

    def body(j, carry):
      off = base + j * K
      pltpu.sync_copy(dst_hbm.at[pl.ds(off, K)], dst_v)
      pltpu.sync_copy(rows_v, acc_sh.at[dst_v], add=True)
      return carry

    lax.fori_loop(0, NCHUNK, body, 0)
    plsc.subcore_barrier()
    pltpu.sync_copy(acc_sh.at[pl.ds(r0, RPT)], out_hbm.at[c, pl.ds(r0, RPT)])

  return deg_kernel


def _make_edge_pass(D):
  @functools.partial(
      pl.kernel,
      out_type=jax.ShapeDtypeStruct((NC, N_NODES, D), jnp.float32),
      mesh=_MESH,
      scratch_types=[
          pltpu.VMEM((K,), jnp.int32),
          pltpu.VMEM((K,), jnp.int32),
          pltpu.VMEM((K, D), jnp.float32),
          pltpu.VMEM_SHARED((N_NODES, D), jnp.float32),
          pltpu.SemaphoreType.DMA,
      ],
  )
  def edge_pass(table_hbm, src_hbm, dst_hbm, zeros_hbm, out_hbm,
                src_v, dst_v, rows_v, acc_sh, sem):
    c = lax.axis_index("c")
    s = lax.axis_index("s")
    wid = c * NS + s
    r0 = s * RPT
    pltpu.sync_copy(zeros_hbm.at[pl.ds(r0, RPT)], acc_sh.at[pl.ds(r0, RPT)])
    plsc.subcore_barrier()
    base = wid * EPW

    def body(j, carry):
      off = base + j * K
      pltpu.sync_copy(src_hbm.at[pl.ds(off, K)], src_v)
      pltpu.sync_copy(dst_hbm.at[pl.ds(off, K)], dst_v)
      pltpu.async_copy(table_hbm.at[src_v], rows_v, sem).wait()
      pltpu.sync_copy(rows_v, acc_sh.at[dst_v], add=True)
      return carry

    lax.fori_loop(0, NCHUNK, body, 0)
    plsc.subcore_barrier()
    pltpu.sync_copy(acc_sh.at[pl.ds(r0, RPT)], out_hbm.at[c, pl.ds(r0, RPT)])

  return edge_pass


def _tc1_body(x_ref, w_ref, degp_ref, hs_ref, dinv_ref):
  deg = degp_ref[0] + degp_ref[1] + 1.0
  dinv = lax.rsqrt(deg)
  dinv_ref[...] = dinv
  u = jnp.dot(x_ref[...], w_ref[...], preferred_element_type=jnp.float32)
  hs_ref[...] = u * dinv


def _tc2_body(accp_ref, hs1_ref, dinv_ref, b1_ref, w2_ref, hs2_ref):
  dinv = dinv_ref[...]
  t = (accp_ref[0] + accp_ref[1] + hs1_ref[...]) * dinv + b1_ref[...]
  h1 = jnp.maximum(t, 0.0)
  u2 = jnp.dot(h1, w2_ref[...], preferred_element_type=jnp.float32)
  hs2_ref[...] = u2 * dinv[:, :8]


def _tc3_body(accp_ref, hs2_ref, dinv_ref, b2_ref, wfc_ref, bfc_ref, out_ref):
  dinv8 = dinv_ref[...][:, :8]
  t = (accp_ref[0] + accp_ref[1] + hs2_ref[...]) * dinv8 + b2_ref[...]
  h2 = jnp.maximum(t, 0.0)
  out_ref[...] = (
      jnp.dot(h2, wfc_ref[...], preferred_element_type=jnp.float32)
      + bfc_ref[...])


_deg_kernel = _make_deg_kernel()
_edge_pass16 = _make_edge_pass(16)
_edge_pass8 = _make_edge_pass(8)

_tc1 = pl.pallas_call(
    _tc1_body,
    out_shape=(jax.ShapeDtypeStruct((N_NODES, 16), jnp.float32),
               jax.ShapeDtypeStruct((N_NODES, 16), jnp.float32)),
)

_tc2 = pl.pallas_call(
    _tc2_body,
    out_shape=jax.ShapeDtypeStruct((N_NODES, 8), jnp.float32),
)

_tc3 = pl.pallas_call(
    _tc3_body,
    out_shape=jax.ShapeDtypeStruct((N_NODES, 128), jnp.float32),
)


def kernel(x, edge_index, W1, b1, W2, b2, Wfc, bfc):
  src = edge_index[0]
  dst = edge_index[1]
  zeros16 = jnp.zeros((N_NODES, 16), jnp.float32)
  zeros8 = jnp.zeros((N_NODES, 8), jnp.float32)

  degp = _deg_kernel(dst, zeros16)
  hs1, dinv = _tc1(x, W1, degp)
  acc1 = _edge_pass16(hs1, src, dst, zeros16)
  hs2 = _tc2(acc1, hs1, dinv, b1.reshape(1, 16), W2)
  acc2 = _edge_pass8(hs2, src, dst, zeros8)
  out = _tc3(acc2, hs2, dinv, b2.reshape(1, 8), Wfc, bfc.reshape(1, 128))
  return out


# speedup vs baseline: 16.3636x; 16.3636x over previous
import functools

import jax
import jax.numpy as jnp
from jax import lax
from jax.experimental import pallas as pl
from jax.experimental.pallas import tpu as pltpu
from jax.experimental.pallas import tpu_sc as plsc

N_NODES = 10000
N_PAD = 10240
N_EDGES = 320000
NC, NS = 2, 16
NW = NC * NS
EPW = N_EDGES // NW
K = 80
NCHUNK = EPW // K
RPT = N_PAD // NS

_MESH = plsc.VectorSubcoreMesh(
    core_axis_name="c", subcore_axis_name="s", num_cores=NC, num_subcores=NS)


def _make_deg_kernel():
  D = 16

  @functools.partial(
      pl.kernel,
      out_type=jax.ShapeDtypeStruct((NC, N_PAD, D), jnp.float32),
      mesh=_MESH,
      compiler_params=pltpu.CompilerParams(use_tc_tiling_on_sc=False),
      scratch_types=[
          pltpu.VMEM((K,), jnp.int32),
          pltpu.VMEM((K, D), jnp.float32),
          pltpu.VMEM_SHARED((N_PAD, D), jnp.float32),
      ],
  )
  def deg_kernel(dst_hbm, zeros_hbm, out_hbm, dst_v, rows_v, acc_sh):
    c = lax.axis_index("c")
    s = lax.axis_index("s")
    wid = c * NS + s
    r0 = s * RPT
    pltpu.sync_copy(zeros_hbm.at[pl.ds(r0, RPT)], acc_sh.at[pl.ds(r0, RPT)])
    ones = jnp.ones((16,), jnp.float32)
    for r in range(K):
      rows_v[r, :] = ones
    plsc.subcore_barrier()
    base = wid * EPW

    def body(j, carry):
      off = base + j * K
      pltpu.sync_copy(dst_hbm.at[pl.ds(off, K)], dst_v)
      pltpu.sync_copy(rows_v, acc_sh.at[dst_v], add=True)
      return carry

    lax.fori_loop(0, NCHUNK, body, 0)
    plsc.subcore_barrier()
    pltpu.sync_copy(acc_sh.at[pl.ds(r0, RPT)], out_hbm.at[c, pl.ds(r0, RPT)])

  return deg_kernel


def _make_edge_pass(D):
  @functools.partial(
      pl.kernel,
      out_type=jax.ShapeDtypeStruct((NC, N_PAD, D), jnp.float32),
      mesh=_MESH,
      compiler_params=pltpu.CompilerParams(use_tc_tiling_on_sc=False),
      scratch_types=[
          pltpu.VMEM((K,), jnp.int32),
          pltpu.VMEM((K,), jnp.int32),
          pltpu.VMEM((K, D), jnp.float32),
          pltpu.VMEM_SHARED((N_PAD, D), jnp.float32),
          pltpu.SemaphoreType.DMA,
      ],
  )
  def edge_pass(table_hbm, src_hbm, dst_hbm, zeros_hbm, out_hbm,
                src_v, dst_v, rows_v, acc_sh, sem):
    c = lax.axis_index("c")
    s = lax.axis_index("s")
    wid = c * NS + s
    r0 = s * RPT
    pltpu.sync_copy(zeros_hbm.at[pl.ds(r0, RPT)], acc_sh.at[pl.ds(r0, RPT)])
    plsc.subcore_barrier()
    base = wid * EPW

    def body(j, carry):
      off = base + j * K
      pltpu.sync_copy(src_hbm.at[pl.ds(off, K)], src_v)
      pltpu.sync_copy(dst_hbm.at[pl.ds(off, K)], dst_v)
      pltpu.async_copy(table_hbm.at[src_v], rows_v, sem).wait()
      pltpu.sync_copy(rows_v, acc_sh.at[dst_v], add=True)
      return carry

    lax.fori_loop(0, NCHUNK, body, 0)
    plsc.subcore_barrier()
    pltpu.sync_copy(acc_sh.at[pl.ds(r0, RPT)], out_hbm.at[c, pl.ds(r0, RPT)])

  return edge_pass


def _tc1_body(x_ref, w_ref, degp_ref, hs_ref, dinv_ref):
  deg = degp_ref[0] + degp_ref[1] + 1.0
  dinv = lax.rsqrt(deg)
  dinv_ref[...] = dinv
  u = jnp.dot(x_ref[...], w_ref[...], preferred_element_type=jnp.float32)
  hs_ref[...] = u * dinv


def _tc2_body(accp_ref, hs1_ref, dinv_ref, b1_ref, w2_ref, hs2_ref):
  dinv = dinv_ref[...]
  t = (accp_ref[0] + accp_ref[1] + hs1_ref[...]) * dinv + b1_ref[...]
  h1 = jnp.maximum(t, 0.0)
  u2 = jnp.dot(h1, w2_ref[...], preferred_element_type=jnp.float32)
  hs2_ref[...] = u2 * dinv[:, :8]


def _tc3_body(accp_ref, hs2_ref, dinv_ref, b2_ref, wfc_ref, bfc_ref, out_ref):
  dinv8 = dinv_ref[...][:, :8]
  t = (accp_ref[0] + accp_ref[1] + hs2_ref[...]) * dinv8 + b2_ref[...]
  h2 = jnp.maximum(t, 0.0)
  out_ref[...] = (
      jnp.dot(h2, wfc_ref[...], preferred_element_type=jnp.float32)
      + bfc_ref[...])


_deg_kernel = _make_deg_kernel()
_edge_pass16 = _make_edge_pass(16)
_edge_pass8 = _make_edge_pass(8)

_tc1 = pl.pallas_call(
    _tc1_body,
    out_shape=(jax.ShapeDtypeStruct((N_NODES, 16), jnp.float32),
               jax.ShapeDtypeStruct((N_NODES, 16), jnp.float32)),
)

_tc2 = pl.pallas_call(
    _tc2_body,
    out_shape=jax.ShapeDtypeStruct((N_NODES, 8), jnp.float32),
)

_tc3 = pl.pallas_call(
    _tc3_body,
    out_shape=jax.ShapeDtypeStruct((N_NODES, 128), jnp.float32),
)


def kernel(x, edge_index, W1, b1, W2, b2, Wfc, bfc):
  src = edge_index[0]
  dst = edge_index[1]
  zeros16 = jnp.zeros((N_PAD, 16), jnp.float32)
  zeros8 = jnp.zeros((N_PAD, 8), jnp.float32)
  pad = ((0, N_PAD - N_NODES), (0, 0))

  degp = _deg_kernel(dst, zeros16)
  hs1, dinv = _tc1(x, W1, degp[:, :N_NODES])
  acc1 = _edge_pass16(jnp.pad(hs1, pad), src, dst, zeros16)
  hs2 = _tc2(acc1[:, :N_NODES], hs1, dinv, b1.reshape(1, 16), W2)
  acc2 = _edge_pass8(jnp.pad(hs2, pad), src, dst, zeros8)
  out = _tc3(acc2[:, :N_NODES], hs2, dinv, b2.reshape(1, 8), Wfc, bfc.reshape(1, 128))
  return out
